# baseline (speedup 1.0000x reference)
import functools

import jax
import jax.numpy as jnp
from jax import lax
from jax.experimental import pallas as pl
from jax.experimental.pallas import tpu as pltpu
from jax.experimental.pallas import tpu_sc as plsc

_B, _C, _I, _S = 32, 128, 9, 256
_D = _I * _S
_R = _B * _C
_NC, _NS = 2, 16
_NW = _NC * _NS
_RPW = _R // _NW
_CH = 16
_NCH = _RPW // _CH


def _row_indices(key):
    perm = jax.random.permutation(key, _C).astype(jnp.int32)
    r = jnp.arange(_R, dtype=jnp.int32)
    return ((r // _C) * _C + perm[r % _C]).reshape(_NW, _NCH, _CH)


_mesh = plsc.VectorSubcoreMesh(core_axis_name="c", subcore_axis_name="s")


@functools.partial(
    pl.kernel,
    mesh=_mesh,
    out_type=[
        jax.ShapeDtypeStruct((_R, _D), jnp.float32),
        jax.ShapeDtypeStruct((_R, _D), jnp.float32),
    ],
    scratch_types=[
        pltpu.VMEM((_NCH, _CH), jnp.int32),
        pltpu.VMEM((_NCH, _CH), jnp.int32),
        pltpu.VMEM((_CH, _D), jnp.float32),
        pltpu.SemaphoreType.DMA,
    ],
)
def _shuffle(a_hbm, b_hbm, ia_hbm, ib_hbm, oa_hbm, ob_hbm,
             ia_v, ib_v, buf, sem):
    wid = lax.axis_index("s") * _NC + lax.axis_index("c")
    base = wid * _RPW
    pltpu.sync_copy(ia_hbm.at[wid], ia_v)
    pltpu.sync_copy(ib_hbm.at[wid], ib_v)
    for j in range(_NCH):
        pltpu.async_copy(a_hbm.at[ia_v.at[j]], buf, sem).wait()
        pltpu.sync_copy(buf, oa_hbm.at[pl.ds(base + j * _CH, _CH)])
    for j in range(_NCH):
        pltpu.async_copy(b_hbm.at[ib_v.at[j]], buf, sem).wait()
        pltpu.sync_copy(buf, ob_hbm.at[pl.ds(base + j * _CH, _CH)])


def kernel(x_shake_audio, x_shake_acc, labels):
    ia = _row_indices(jax.random.key(1))
    ib = _row_indices(jax.random.key(2))
    a = x_shake_audio.reshape(_R, _D)
    b = x_shake_acc.reshape(_R, _D)
    oa, ob = _shuffle(a, b, ia, ib)
    return (oa.reshape(_B, _C, _I, _S), ob.reshape(_B, _C, _I, _S), labels)


# speedup vs baseline: 1.0318x; 1.0318x over previous
import functools

import jax
import jax.numpy as jnp
from jax import lax
from jax.experimental import pallas as pl
from jax.experimental.pallas import tpu as pltpu
from jax.experimental.pallas import tpu_sc as plsc

_B, _C, _I, _S = 32, 128, 9, 256
_D = _I * _S
_R = _B * _C
_NC, _NS = 2, 16
_NW = _NC * _NS
_RPW = _R // _NW
_CH = 16
_NCH = _RPW // _CH


def _row_indices(key):
    perm = jax.random.permutation(key, _C).astype(jnp.int32)
    r = jnp.arange(_R, dtype=jnp.int32)
    return ((r // _C) * _C + perm[r % _C]).reshape(_NW, _NCH, _CH)


_mesh = plsc.VectorSubcoreMesh(core_axis_name="c", subcore_axis_name="s")

_NB = 3
_NCHT = 2 * _NCH


@functools.partial(
    pl.kernel,
    mesh=_mesh,
    out_type=[
        jax.ShapeDtypeStruct((_R, _D), jnp.float32),
        jax.ShapeDtypeStruct((_R, _D), jnp.float32),
    ],
    scratch_types=[
        pltpu.VMEM((_NCH, _CH), jnp.int32),
        pltpu.VMEM((_NCH, _CH), jnp.int32),
        *([pltpu.VMEM((_CH, _D), jnp.float32)] * _NB),
        *([pltpu.SemaphoreType.DMA] * _NB),
        *([pltpu.SemaphoreType.DMA] * _NB),
    ],
)
def _shuffle(a_hbm, b_hbm, ia_hbm, ib_hbm, oa_hbm, ob_hbm,
             ia_v, ib_v, *rest):
    bufs = rest[:_NB]
    gsem = rest[_NB:2 * _NB]
    ssem = rest[2 * _NB:3 * _NB]
    wid = lax.axis_index("s") * _NC + lax.axis_index("c")
    base = wid * _RPW
    pltpu.sync_copy(ia_hbm.at[wid], ia_v)
    pltpu.sync_copy(ib_hbm.at[wid], ib_v)

    def gather(k):
        src = a_hbm if k < _NCH else b_hbm
        idx = ia_v.at[k] if k < _NCH else ib_v.at[k - _NCH]
        return pltpu.async_copy(src.at[idx], bufs[k % _NB], gsem[k % _NB])

    def store(k):
        j = k if k < _NCH else k - _NCH
        dst = oa_hbm if k < _NCH else ob_hbm
        return pltpu.async_copy(
            bufs[k % _NB], dst.at[pl.ds(base + j * _CH, _CH)], ssem[k % _NB])

    gcp = [None] * _NCHT
    scp = [None] * _NCHT
    for k in range(_NB - 1):
        gcp[k] = gather(k)
    for k in range(_NCHT):
        j = k + _NB - 1
        if j < _NCHT:
            if j - _NB >= 0:
                scp[j - _NB].wait()
            gcp[j] = gather(j)
        gcp[k].wait()
        scp[k] = store(k)
    for k in range(_NCHT - _NB, _NCHT):
        scp[k].wait()


def kernel(x_shake_audio, x_shake_acc, labels):
    ia = _row_indices(jax.random.key(1))
    ib = _row_indices(jax.random.key(2))
    a = x_shake_audio.reshape(_R, _D)
    b = x_shake_acc.reshape(_R, _D)
    oa, ob = _shuffle(a, b, ia, ib)
    return (oa.reshape(_B, _C, _I, _S), ob.reshape(_B, _C, _I, _S), labels)


# speedup vs baseline: 1.5010x; 1.4547x over previous
import functools

import jax
import jax.numpy as jnp
import numpy as np
from jax import lax
from jax.experimental import pallas as pl
from jax.experimental.pallas import tpu as pltpu
from jax.experimental.pallas import tpu_sc as plsc

_B, _C, _I, _S = 32, 128, 9, 256
_NC, _NS = 2, 16
_NW = _NC * _NS
_CPW = _C // _NW
_GB = 8
_NBG = _B // _GB
_NB = 3
_NCHT = 2 * _CPW * _NBG

_PERM_A = np.array([
    19, 76, 118, 54, 90, 30, 7, 96, 121, 115, 6, 35, 23, 58, 16, 21,
    77, 94, 116, 61, 38, 3, 105, 81, 26, 32, 64, 37, 56, 51, 2, 122,
    63, 52, 20, 89, 95, 44, 47, 123, 79, 84, 50, 78, 72, 83, 42, 62,
    69, 53, 0, 8, 109, 22, 13, 29, 99, 110, 34, 70, 18, 103, 86, 75,
    91, 111, 24, 113, 1, 65, 48, 5, 45, 49, 33, 74, 55, 60, 119, 57,
    124, 27, 112, 10, 93, 68, 15, 73, 40, 67, 88, 102, 107, 66, 80, 100,
    120, 71, 17, 59, 98, 108, 114, 36, 125, 101, 92, 28, 46, 9, 104, 117,
    4, 12, 87, 85, 14, 82, 31, 106, 127, 126, 97, 41, 25, 43, 39, 11,
], dtype=np.int32)
_PERM_B = np.array([
    83, 107, 91, 52, 58, 2, 59, 73, 15, 53, 63, 10, 113, 67, 29, 94,
    62, 92, 117, 126, 110, 98, 81, 66, 55, 9, 85, 28, 124, 80, 18, 45,
    31, 114, 12, 61, 51, 102, 14, 93, 104, 4, 115, 95, 32, 118, 68, 7,
    79, 88, 43, 26, 111, 77, 69, 17, 22, 119, 101, 87, 108, 60, 109, 13,
    112, 1, 21, 72, 38, 106, 89, 70, 24, 122, 100, 49, 116, 25, 50, 48,
    6, 20, 120, 82, 90, 33, 35, 11, 39, 71, 76, 47, 127, 74, 103, 96,
    37, 105, 16, 56, 27, 40, 86, 123, 44, 30, 64, 99, 34, 19, 121, 78,
    23, 3, 46, 65, 42, 8, 36, 57, 54, 97, 41, 125, 5, 84, 0, 75,
], dtype=np.int32)


def _select_tree(vals, wid, bit=4):
    if bit < 0:
        return jnp.int32(int(vals[0]))
    half = 1 << bit
    lo = _select_tree(vals[:half], wid, bit - 1)
    hi = _select_tree(vals[half:], wid, bit - 1)
    return jnp.where((wid >> bit) & 1 == 1, hi, lo)


_mesh = plsc.VectorSubcoreMesh(core_axis_name="c", subcore_axis_name="s")


@functools.partial(
    pl.kernel,
    mesh=_mesh,
    out_type=[
        jax.ShapeDtypeStruct((_B, _C, _I, _S), jnp.float32),
        jax.ShapeDtypeStruct((_B, _C, _I, _S), jnp.float32),
    ],
    scratch_types=[
        *([pltpu.VMEM((_GB, 1, _I, _S), jnp.float32)] * _NB),
        *([pltpu.SemaphoreType.DMA] * _NB),
        *([pltpu.SemaphoreType.DMA] * _NB),
    ],
    compiler_params=pltpu.CompilerParams(use_tc_tiling_on_sc=True),
)
def _shuffle(a_hbm, b_hbm, oa_hbm, ob_hbm, *rest):
    bufs = rest[:_NB]
    gsem = rest[_NB:2 * _NB]
    ssem = rest[2 * _NB:3 * _NB]
    wid = lax.axis_index("s") * _NC + lax.axis_index("c")
    c0 = wid * _CPW
    src_c = [
        [_select_tree(perm[ci::_CPW], wid) for ci in range(_CPW)]
        for perm in (_PERM_A, _PERM_B)
    ]

    def chunk(k):
        t, r = divmod(k, _CPW * _NBG)
        ci, bg = divmod(r, _NBG)
        return t, ci, bg

    def gather(k):
        t, ci, bg = chunk(k)
        src = a_hbm if t == 0 else b_hbm
        return pltpu.async_copy(
            src.at[pl.ds(bg * _GB, _GB), pl.ds(src_c[t][ci], 1)],
            bufs[k % _NB], gsem[k % _NB])

    def store(k):
        t, ci, bg = chunk(k)
        dst = oa_hbm if t == 0 else ob_hbm
        return pltpu.async_copy(
            bufs[k % _NB],
            dst.at[pl.ds(bg * _GB, _GB), pl.ds(c0 + ci, 1)],
            ssem[k % _NB])

    gcp = [None] * _NCHT
    scp = [None] * _NCHT
    for k in range(_NB - 1):
        gcp[k] = gather(k)
    for k in range(_NCHT):
        j = k + _NB - 1
        if j < _NCHT:
            if j - _NB >= 0:
                scp[j - _NB].wait()
            gcp[j] = gather(j)
        gcp[k].wait()
        scp[k] = store(k)
    for k in range(_NCHT - _NB, _NCHT):
        scp[k].wait()


def kernel(x_shake_audio, x_shake_acc, labels):
    oa, ob = _shuffle(x_shake_audio, x_shake_acc)
    return (oa, ob, labels)


# speedup vs baseline: 4.8920x; 3.2591x over previous
import functools

import jax
import jax.numpy as jnp
import numpy as np
from jax import lax
from jax.experimental import pallas as pl
from jax.experimental.pallas import tpu as pltpu
from jax.experimental.pallas import tpu_sc as plsc

_B, _C, _I, _S = 32, 128, 9, 256
_P = _B * _I
_R = _P * _C
_NC, _NS = 2, 16
_NW = _NC * _NS
_PPW = _P // _NW
_NB = 3
_NCHT = 2 * _PPW
_L = 16

_PERM_A = np.array([
    19, 76, 118, 54, 90, 30, 7, 96, 121, 115, 6, 35, 23, 58, 16, 21,
    77, 94, 116, 61, 38, 3, 105, 81, 26, 32, 64, 37, 56, 51, 2, 122,
    63, 52, 20, 89, 95, 44, 47, 123, 79, 84, 50, 78, 72, 83, 42, 62,
    69, 53, 0, 8, 109, 22, 13, 29, 99, 110, 34, 70, 18, 103, 86, 75,
    91, 111, 24, 113, 1, 65, 48, 5, 45, 49, 33, 74, 55, 60, 119, 57,
    124, 27, 112, 10, 93, 68, 15, 73, 40, 67, 88, 102, 107, 66, 80, 100,
    120, 71, 17, 59, 98, 108, 114, 36, 125, 101, 92, 28, 46, 9, 104, 117,
    4, 12, 87, 85, 14, 82, 31, 106, 127, 126, 97, 41, 25, 43, 39, 11,
], dtype=np.int32)
_PERM_B = np.array([
    83, 107, 91, 52, 58, 2, 59, 73, 15, 53, 63, 10, 113, 67, 29, 94,
    62, 92, 117, 126, 110, 98, 81, 66, 55, 9, 85, 28, 124, 80, 18, 45,
    31, 114, 12, 61, 51, 102, 14, 93, 104, 4, 115, 95, 32, 118, 68, 7,
    79, 88, 43, 26, 111, 77, 69, 17, 22, 119, 101, 87, 108, 60, 109, 13,
    112, 1, 21, 72, 38, 106, 89, 70, 24, 122, 100, 49, 116, 25, 50, 48,
    6, 20, 120, 82, 90, 33, 35, 11, 39, 71, 76, 47, 127, 74, 103, 96,
    37, 105, 16, 56, 27, 40, 86, 123, 44, 30, 64, 99, 34, 19, 121, 78,
    23, 3, 46, 65, 42, 8, 36, 57, 54, 97, 41, 125, 5, 84, 0, 75,
], dtype=np.int32)

_mesh = plsc.VectorSubcoreMesh(core_axis_name="c", subcore_axis_name="s")


@functools.partial(
    pl.kernel,
    mesh=_mesh,
    out_type=[
        jax.ShapeDtypeStruct((_R, _S), jnp.float32),
        jax.ShapeDtypeStruct((_R, _S), jnp.float32),
    ],
    scratch_types=[
        pltpu.VMEM((_C,), jnp.int32),
        pltpu.VMEM((_C,), jnp.int32),
        *([pltpu.VMEM((_C,), jnp.int32)] * _NB),
        *([pltpu.VMEM((_C, _S), jnp.float32)] * _NB),
        *([pltpu.SemaphoreType.DMA] * _NB),
        *([pltpu.SemaphoreType.DMA] * _NB),
    ],
    compiler_params=pltpu.CompilerParams(use_tc_tiling_on_sc=True),
)
def _shuffle(a_hbm, b_hbm, pa_hbm, pb_hbm, oa_hbm, ob_hbm,
             pa_v, pb_v, *rest):
    idxs = rest[:_NB]
    bufs = rest[_NB:2 * _NB]
    gsem = rest[2 * _NB:3 * _NB]
    ssem = rest[3 * _NB:4 * _NB]
    wid = lax.axis_index("s") * _NC + lax.axis_index("c")
    p0 = wid * _PPW
    pltpu.sync_copy(pa_hbm, pa_v)
    pltpu.sync_copy(pb_hbm, pb_v)

    def chunk(k):
        t, j = divmod(k, _PPW)
        return t, p0 + j

    def gather(k):
        t, p = chunk(k)
        src = a_hbm if t == 0 else b_hbm
        perm_v = pa_v if t == 0 else pb_v
        idx = idxs[k % _NB]
        base = p * _C
        for v in range(_C // _L):
            idx[pl.ds(v * _L, _L)] = base + perm_v[pl.ds(v * _L, _L)]
        return pltpu.async_copy(src.at[idx], bufs[k % _NB], gsem[k % _NB])

    def store(k):
        t, p = chunk(k)
        dst = oa_hbm if t == 0 else ob_hbm
        return pltpu.async_copy(
            bufs[k % _NB], dst.at[pl.ds(p * _C, _C)], ssem[k % _NB])

    gcp = [None] * _NCHT
    scp = [None] * _NCHT
    for k in range(_NB - 1):
        gcp[k] = gather(k)
    for k in range(_NCHT):
        j = k + _NB - 1
        if j < _NCHT:
            if j - _NB >= 0:
                scp[j - _NB].wait()
            gcp[j] = gather(j)
        gcp[k].wait()
        scp[k] = store(k)
    for k in range(_NCHT - _NB, _NCHT):
        scp[k].wait()


def kernel(x_shake_audio, x_shake_acc, labels):
    a = x_shake_audio.transpose(0, 2, 1, 3).reshape(_R, _S)
    b = x_shake_acc.transpose(0, 2, 1, 3).reshape(_R, _S)
    oa, ob = _shuffle(a, b, jnp.asarray(_PERM_A), jnp.asarray(_PERM_B))
    oa = oa.reshape(_B, _I, _C, _S).transpose(0, 2, 1, 3)
    ob = ob.reshape(_B, _I, _C, _S).transpose(0, 2, 1, 3)
    return (oa, ob, labels)


# speedup vs baseline: 4.8947x; 1.0006x over previous
import functools

import jax
import jax.numpy as jnp
import numpy as np
from jax import lax
from jax.experimental import pallas as pl
from jax.experimental.pallas import tpu as pltpu
from jax.experimental.pallas import tpu_sc as plsc

_B, _C, _I, _S = 32, 128, 9, 256
_P = _B * _I
_R = _P * _C
_NC, _NS = 2, 16
_NW = _NC * _NS
_PPW = _P // _NW
_HC = _C // 2
_NB = 6
_NCHT = 2 * _PPW * 2
_L = 16

_PERM_A = np.array([
    19, 76, 118, 54, 90, 30, 7, 96, 121, 115, 6, 35, 23, 58, 16, 21,
    77, 94, 116, 61, 38, 3, 105, 81, 26, 32, 64, 37, 56, 51, 2, 122,
    63, 52, 20, 89, 95, 44, 47, 123, 79, 84, 50, 78, 72, 83, 42, 62,
    69, 53, 0, 8, 109, 22, 13, 29, 99, 110, 34, 70, 18, 103, 86, 75,
    91, 111, 24, 113, 1, 65, 48, 5, 45, 49, 33, 74, 55, 60, 119, 57,
    124, 27, 112, 10, 93, 68, 15, 73, 40, 67, 88, 102, 107, 66, 80, 100,
    120, 71, 17, 59, 98, 108, 114, 36, 125, 101, 92, 28, 46, 9, 104, 117,
    4, 12, 87, 85, 14, 82, 31, 106, 127, 126, 97, 41, 25, 43, 39, 11,
], dtype=np.int32)
_PERM_B = np.array([
    83, 107, 91, 52, 58, 2, 59, 73, 15, 53, 63, 10, 113, 67, 29, 94,
    62, 92, 117, 126, 110, 98, 81, 66, 55, 9, 85, 28, 124, 80, 18, 45,
    31, 114, 12, 61, 51, 102, 14, 93, 104, 4, 115, 95, 32, 118, 68, 7,
    79, 88, 43, 26, 111, 77, 69, 17, 22, 119, 101, 87, 108, 60, 109, 13,
    112, 1, 21, 72, 38, 106, 89, 70, 24, 122, 100, 49, 116, 25, 50, 48,
    6, 20, 120, 82, 90, 33, 35, 11, 39, 71, 76, 47, 127, 74, 103, 96,
    37, 105, 16, 56, 27, 40, 86, 123, 44, 30, 64, 99, 34, 19, 121, 78,
    23, 3, 46, 65, 42, 8, 36, 57, 54, 97, 41, 125, 5, 84, 0, 75,
], dtype=np.int32)

_mesh = plsc.VectorSubcoreMesh(core_axis_name="c", subcore_axis_name="s")


@functools.partial(
    pl.kernel,
    mesh=_mesh,
    out_type=[
        jax.ShapeDtypeStruct((_R, _S), jnp.float32),
        jax.ShapeDtypeStruct((_R, _S), jnp.float32),
    ],
    scratch_types=[
        pltpu.VMEM((_C,), jnp.int32),
        pltpu.VMEM((_C,), jnp.int32),
        *([pltpu.VMEM((_HC,), jnp.int32)] * _NB),
        *([pltpu.VMEM((_HC, _S), jnp.float32)] * _NB),
        *([pltpu.SemaphoreType.DMA] * _NB),
        *([pltpu.SemaphoreType.DMA] * _NB),
    ],
    compiler_params=pltpu.CompilerParams(use_tc_tiling_on_sc=True),
)
def _shuffle(a_hbm, b_hbm, pa_hbm, pb_hbm, oa_hbm, ob_hbm,
             pa_v, pb_v, *rest):
    idxs = rest[:_NB]
    bufs = rest[_NB:2 * _NB]
    gsem = rest[2 * _NB:3 * _NB]
    ssem = rest[3 * _NB:4 * _NB]
    wid = lax.axis_index("s") * _NC + lax.axis_index("c")
    p0 = wid * _PPW
    pltpu.sync_copy(pa_hbm, pa_v)
    pltpu.sync_copy(pb_hbm, pb_v)

    def chunk(k):
        t, j = divmod(k, _PPW * 2)
        pj, h = divmod(j, 2)
        return t, p0 + pj, h

    def gather(k):
        t, p, h = chunk(k)
        src = a_hbm if t == 0 else b_hbm
        perm_v = pa_v if t == 0 else pb_v
        idx = idxs[k % _NB]
        base = p * _C
        for v in range(_HC // _L):
            idx[pl.ds(v * _L, _L)] = base + perm_v[
                pl.ds(h * _HC + v * _L, _L)]
        return pltpu.async_copy(src.at[idx], bufs[k % _NB], gsem[k % _NB])

    def store(k):
        t, p, h = chunk(k)
        dst = oa_hbm if t == 0 else ob_hbm
        return pltpu.async_copy(
            bufs[k % _NB], dst.at[pl.ds(p * _C + h * _HC, _HC)],
            ssem[k % _NB])

    gcp = [None] * _NCHT
    scp = [None] * _NCHT
    for k in range(_NB - 1):
        gcp[k] = gather(k)
    for k in range(_NCHT):
        j = k + _NB - 1
        if j < _NCHT:
            if j - _NB >= 0:
                scp[j - _NB].wait()
            gcp[j] = gather(j)
        gcp[k].wait()
        scp[k] = store(k)
    for k in range(_NCHT - _NB, _NCHT):
        scp[k].wait()


def kernel(x_shake_audio, x_shake_acc, labels):
    a = x_shake_audio.transpose(0, 2, 1, 3).reshape(_R, _S)
    b = x_shake_acc.transpose(0, 2, 1, 3).reshape(_R, _S)
    oa, ob = _shuffle(a, b, jnp.asarray(_PERM_A), jnp.asarray(_PERM_B))
    oa = oa.reshape(_B, _I, _C, _S).transpose(0, 2, 1, 3)
    ob = ob.reshape(_B, _I, _C, _S).transpose(0, 2, 1, 3)
    return (oa, ob, labels)
